# baseline (speedup 1.0000x reference)
import functools

import jax
import jax.numpy as jnp
from jax import lax
from jax.experimental import pallas as pl
from jax.experimental.pallas import tpu as pltpu
from jax.experimental.pallas import tpu_sc as plsc

_N = 10000
_D = 128
_E = 320000
_NPAD = 10240
_K = 64
_NCH = 324
_NCHH = 324
_EPT = _NCH * _K
_EPAD = 16 * _EPT
_RPT = _NPAD // 16
_ZR = 16
_NST = _NCH // _NCHH
_Q = 32
_QP = _Q + 16

_pallas_call = pl.pallas_call
_pl_kernel = pl.kernel


def _mm_epilogue(h, as_ref, ad_ref, h_ref, a2_ref):
    h_ref[...] = h
    a2_ref[0, :] = jnp.sum(h * as_ref[0, :], axis=1)
    a2_ref[1, :] = jnp.sum(h * ad_ref[0, :], axis=1)


def _tc0_body(x_ref, w_ref, as_ref, ad_ref, h_ref, a2_ref):
    h = jnp.dot(x_ref[...], w_ref[...], preferred_element_type=jnp.float32)
    _mm_epilogue(h, as_ref, ad_ref, h_ref, a2_ref)


def _tc0(x, W, a_s, a_d):
    blk = 512
    out = W.shape[1]
    return _pallas_call(
        _tc0_body,
        grid=(_NPAD // blk,),
        in_specs=[
            pl.BlockSpec((blk, _D), lambda i: (i, 0)),
            pl.BlockSpec((_D, out), lambda i: (0, 0)),
            pl.BlockSpec((1, out), lambda i: (0, 0)),
            pl.BlockSpec((1, out), lambda i: (0, 0)),
        ],
        out_specs=[
            pl.BlockSpec((blk, out), lambda i: (i, 0)),
            pl.BlockSpec((2, blk), lambda i: (0, i)),
        ],
        out_shape=[
            jax.ShapeDtypeStruct((_NPAD, out), jnp.float32),
            jax.ShapeDtypeStruct((2, _NPAD), jnp.float32),
        ],
    )(x, W, a_s[None], a_d[None])


def _quadrants(acc_ref, nq):
    lanes = lax.broadcasted_iota(jnp.int32, acc_ref[0].shape, 1)
    parts = []
    for q in range(nq):
        s = acc_ref[q]
        den = jnp.sum(jnp.where(lanes == _Q, s, 0.0), axis=1, keepdims=True)
        parts.append(s[:, :_Q] / (den + 1e-16))
    return jnp.concatenate(parts, axis=1)


def _tcc_body(acc_ref, b_ref, w_ref, as_ref, ad_ref, h_ref, a2_ref):
    msg = _quadrants(acc_ref, 4) + b_ref[0, :]
    xact = msg * jax.nn.sigmoid(msg)
    h = jnp.dot(xact, w_ref[...], preferred_element_type=jnp.float32)
    _mm_epilogue(h, as_ref, ad_ref, h_ref, a2_ref)


def _tc_comb(acc, b_prev, W, a_s, a_d):
    blk = 512
    oprev, out = W.shape
    return _pallas_call(
        _tcc_body,
        grid=(_NPAD // blk,),
        in_specs=[
            pl.BlockSpec((4, blk, _QP), lambda i: (0, i, 0)),
            pl.BlockSpec((1, oprev), lambda i: (0, 0)),
            pl.BlockSpec((oprev, out), lambda i: (0, 0)),
            pl.BlockSpec((1, out), lambda i: (0, 0)),
            pl.BlockSpec((1, out), lambda i: (0, 0)),
        ],
        out_specs=[
            pl.BlockSpec((blk, out), lambda i: (i, 0)),
            pl.BlockSpec((2, blk), lambda i: (0, i)),
        ],
        out_shape=[
            jax.ShapeDtypeStruct((_NPAD, out), jnp.float32),
            jax.ShapeDtypeStruct((2, _NPAD), jnp.float32),
        ],
    )(acc, b_prev[None], W, a_s[None], a_d[None])


def _tcf_body(acc_ref, b_ref, o_ref):
    msg = _quadrants(acc_ref, 2) + b_ref[0, :]
    mx = jnp.max(msg, axis=1, keepdims=True)
    t = msg - mx
    o_ref[...] = t - jnp.log(jnp.sum(jnp.exp(t), axis=1, keepdims=True))


def _tc_final(acc, b):
    blk = 400
    out = b.shape[0]
    return _pallas_call(
        _tcf_body,
        grid=(_N // blk,),
        in_specs=[
            pl.BlockSpec((4, blk, _QP), lambda i: (0, i, 0)),
            pl.BlockSpec((1, out), lambda i: (0, 0)),
        ],
        out_specs=pl.BlockSpec((blk, out), lambda i: (i, 0)),
        out_shape=jax.ShapeDtypeStruct((_N, out), jnp.float32),
    )(acc, b[None])


def _sc_gat(h4, a2, s2, d2):
    mesh = plsc.VectorSubcoreMesh(core_axis_name="c", subcore_axis_name="s")

    @functools.partial(
        _pl_kernel,
        out_type=jax.ShapeDtypeStruct((4, _NPAD, _QP), jnp.float32),
        mesh=mesh,
        compiler_params=pltpu.CompilerParams(
            needs_layout_passes=False, use_tc_tiling_on_sc=False),
        scratch_types=[
            pltpu.VMEM((_NCHH, _K), jnp.int32),
            pltpu.VMEM((_NCHH, _K), jnp.int32),
            pltpu.VMEM((_EPT,), jnp.float32),
            pltpu.VMEM((_NPAD,), jnp.float32),
            pltpu.VMEM((_NPAD,), jnp.float32),
            pltpu.VMEM((16,), jnp.float32),
            pltpu.VMEM((16, 16), jnp.float32),
            pltpu.VMEM((4, _K, _Q), jnp.float32),
            pltpu.VMEM((2, _K, _QP), jnp.float32),
            pltpu.VMEM((_ZR, _QP), jnp.float32),
            pltpu.VMEM_SHARED((_NPAD, _QP), jnp.float32),
            pltpu.VMEM_SHARED((16, 16), jnp.float32),
            pltpu.SemaphoreType.DMA,
            pltpu.SemaphoreType.DMA,
            pltpu.SemaphoreType.DMA,
            pltpu.SemaphoreType.DMA,
            pltpu.SemaphoreType.DMA,
            pltpu.SemaphoreType.DMA,
        ],
    )
    def k(h_hbm, a2_hbm, s_hbm, d_hbm, acc_hbm,
          sidx_v, didx_v, ebuf, av, bv, svec, mball, gbuf, hbuf,
          zrow, acc_sh, m_sh, sem0, sem1, sem2, sem3, ssem0, ssem1):
        cid = lax.axis_index("c")
        sid = lax.axis_index("s")

        pltpu.sync_copy(a2_hbm.at[0], av)
        pltpu.sync_copy(a2_hbm.at[1], bv)

        def stage_half(hh):
            pltpu.sync_copy(s_hbm.at[sid, pl.ds(hh * _NCHH, _NCHH)], sidx_v)
            pltpu.sync_copy(d_hbm.at[sid, pl.ds(hh * _NCHH, _NCHH)], didx_v)

        zero = jnp.zeros((16,), jnp.float32)
        for r in range(_ZR):
            for c in range(_QP // 16):
                zrow[r, pl.ds(c * 16, 16)] = zero

        def zero_acc():
            for t in range(_RPT // _ZR):
                pltpu.sync_copy(zrow,
                                acc_sh.at[pl.ds(sid * _RPT + t * _ZR, _ZR)])

        zero_acc()

        stage_half(0)

        def score_body(j, macc):
            m = macc
            for g in range(_K // 16):
                sv = sidx_v[j, pl.ds(g * 16, 16)]
                dv = didx_v[j, pl.ds(g * 16, 16)]
                t = plsc.load_gather(av, [sv]) + plsc.load_gather(bv, [dv])
                e = jnp.maximum(t, 0.2 * t)
                ebuf[pl.ds(j * _K + g * 16, 16)] = e
                sidx_v[j, pl.ds(g * 16, 16)] = sv * 4 + 2 * cid
                m = jnp.maximum(m, e)
            return m

        macc = lax.fori_loop(0, _NCH, score_body,
                             jnp.full((16,), -1e30, jnp.float32))

        svec[...] = macc
        pltpu.sync_copy(svec, m_sh.at[sid])
        plsc.subcore_barrier()
        pltpu.sync_copy(m_sh, mball)
        v = mball[0, :]
        for r in range(1, 16):
            v = jnp.maximum(v, mball[r, :])
        svec[...] = plsc.cummax(v)
        msp = plsc.load_gather(svec, [jnp.full((16,), 15, jnp.int32)])

        def exp_body(g, carry):
            e = ebuf[pl.ds(g * 16, 16)]
            ebuf[pl.ds(g * 16, 16)] = jnp.exp(e - msp)
            return carry

        lax.fori_loop(0, _EPT // 16, exp_body, 0)

        def issue_gather(j, b, sem):
            pltpu.async_copy(h_hbm.at[sidx_v.at[j]], gbuf.at[b], sem)

        lane0 = lax.iota(jnp.int32, 16) == 0

        def wait_scatter(b, ssem):
            pltpu.make_async_copy(hbuf.at[b], acc_sh.at[pl.ds(0, _K)],
                                  ssem).wait()

        def bump_body(j, carry):
            for g in range(_K // 16):
                sl = pl.ds(g * 16, 16)
                sidx_v[j, sl] = sidx_v[j, sl] + 1
            return carry

        for p in range(2):
            if p == 1:
                lax.fori_loop(0, _NCH, bump_body, 0)
                zero_acc()
                plsc.subcore_barrier()
            q = 2 * cid + p
            gsems = (sem0, sem1, sem2, sem3)
            ssems = (ssem0, ssem1)

            def agg_body(jj, carry):
                for b in range(4):
                    j = jj * 4 + b
                    sem = gsems[b]
                    sb = b % 2
                    ssem = ssems[sb]
                    pltpu.make_async_copy(h_hbm.at[pl.ds(0, _K)],
                                          gbuf.at[b], sem).wait()

                    @pl.when(j >= 2)
                    def _():
                        wait_scatter(sb, ssem)

                    for r in range(_K):
                        zr = plsc.load_gather(
                            ebuf, [jnp.full((16,), j * _K + r, jnp.int32)])
                        for c in range(_Q // 16):
                            hbuf[sb, r, pl.ds(c * 16, 16)] = (
                                gbuf[b, r, pl.ds(c * 16, 16)] * zr)
                        hbuf[sb, r, pl.ds(_Q, 16)] = jnp.where(
                            lane0, zr, 0.0)
                    pltpu.async_copy(hbuf.at[sb], acc_sh.at[didx_v.at[j]],
                                     ssem, add=True)

                    @pl.when(j + 4 < _NCH)
                    def _():
                        issue_gather(j + 4, b, sem)
                return carry

            for b in range(4):
                issue_gather(b, b, gsems[b])
            lax.fori_loop(0, _NCH // 4, agg_body, 0)
            wait_scatter(0, ssem0)
            wait_scatter(1, ssem1)
            plsc.subcore_barrier()
            pltpu.sync_copy(acc_sh.at[pl.ds(sid * _RPT, _RPT)],
                            acc_hbm.at[q, pl.ds(sid * _RPT, _RPT)])

    return k(h4, a2, s2, d2)


def kernel(x, edge_index, W0, b0, as0, ad0, W1, b1, as1, ad1,
           W2, b2, as2, ad2):
    src = edge_index[0].astype(jnp.int32)
    dst = edge_index[1].astype(jnp.int32)
    loop = jnp.arange(_N, dtype=jnp.int32)
    pad = jnp.full((_EPAD - _E - _N,), _N, jnp.int32)
    s_all = jnp.concatenate([src, loop, pad])
    d_all = jnp.concatenate([dst, loop, pad])
    s2 = s_all.reshape(16, _NCH, _K)
    d2 = d_all.reshape(16, _NCH, _K)
    xp = jnp.zeros((_NPAD, _D), jnp.float32).at[:_N].set(x)

    W2p = jnp.zeros((_D, _D), jnp.float32).at[:, :W2.shape[1]].set(W2)
    as2p = jnp.zeros((_D,), jnp.float32).at[:as2.shape[0]].set(as2)
    ad2p = jnp.zeros((_D,), jnp.float32).at[:ad2.shape[0]].set(ad2)
    Ws = jnp.stack([W1, W2p, W2p])
    bs = jnp.stack([b0, b1, b1])
    ass = jnp.stack([as1, as2p, as2p])
    ads = jnp.stack([ad1, ad2p, ad2p])

    h, a2 = _tc0(xp, W0, as0, ad0)

    def body(carry, ws):
        h, a2, _ = carry
        acc = _sc_gat(h.reshape(4 * _NPAD, _Q), a2, s2, d2)
        W, b, a_s, a_d = ws
        h2, a22 = _tc_comb(acc, b, W, a_s, a_d)
        return (h2, a22, acc), None

    acc0 = jnp.zeros((4, _NPAD, _QP), jnp.float32)
    (_, _, acc), _ = lax.scan(body, (h, a2, acc0), (Ws, bs, ass, ads))
    return _tc_final(acc, b2)


# speedup vs baseline: 1.2689x; 1.2689x over previous
import functools

import jax
import jax.numpy as jnp
from jax import lax
from jax.experimental import pallas as pl
from jax.experimental.pallas import tpu as pltpu
from jax.experimental.pallas import tpu_sc as plsc

_N = 10000
_D = 128
_E = 320000
_NPAD = 10240
_K = 64
_NCH = 324
_NCHH = 324
_EPT = _NCH * _K
_EPAD = 16 * _EPT
_RPT = _NPAD // 16
_ZR = 64
_NST = _NCH // _NCHH
_Q = 32
_QP = _Q + 16

_pallas_call = pl.pallas_call
_pl_kernel = pl.kernel


def _mm_epilogue(h, as_ref, ad_ref, h_ref, a2_ref):
    h_ref[...] = h
    a2_ref[0, :] = jnp.sum(h * as_ref[0, :], axis=1)
    a2_ref[1, :] = jnp.sum(h * ad_ref[0, :], axis=1)


def _tc0_body(x_ref, w_ref, as_ref, ad_ref, h_ref, a2_ref):
    h = jnp.dot(x_ref[...], w_ref[...], preferred_element_type=jnp.float32)
    _mm_epilogue(h, as_ref, ad_ref, h_ref, a2_ref)


def _tc0(x, W, a_s, a_d):
    blk = 512
    out = W.shape[1]
    return _pallas_call(
        _tc0_body,
        grid=(_NPAD // blk,),
        in_specs=[
            pl.BlockSpec((blk, _D), lambda i: (i, 0)),
            pl.BlockSpec((_D, out), lambda i: (0, 0)),
            pl.BlockSpec((1, out), lambda i: (0, 0)),
            pl.BlockSpec((1, out), lambda i: (0, 0)),
        ],
        out_specs=[
            pl.BlockSpec((blk, out), lambda i: (i, 0)),
            pl.BlockSpec((2, blk), lambda i: (0, i)),
        ],
        out_shape=[
            jax.ShapeDtypeStruct((_NPAD, out), jnp.float32),
            jax.ShapeDtypeStruct((2, _NPAD), jnp.float32),
        ],
    )(x, W, a_s[None], a_d[None])


def _quadrants(acc_ref, nq):
    lanes = lax.broadcasted_iota(jnp.int32, acc_ref[0].shape, 1)
    parts = []
    for q in range(nq):
        s = acc_ref[q]
        den = jnp.sum(jnp.where(lanes == _Q, s, 0.0), axis=1, keepdims=True)
        parts.append(s[:, :_Q] / (den + 1e-16))
    return jnp.concatenate(parts, axis=1)


def _tcc_body(acc_ref, b_ref, w_ref, as_ref, ad_ref, h_ref, a2_ref):
    msg = _quadrants(acc_ref, 4) + b_ref[0, :]
    xact = msg * jax.nn.sigmoid(msg)
    h = jnp.dot(xact, w_ref[...], preferred_element_type=jnp.float32)
    _mm_epilogue(h, as_ref, ad_ref, h_ref, a2_ref)


def _tc_comb(acc, b_prev, W, a_s, a_d):
    blk = 512
    oprev, out = W.shape
    return _pallas_call(
        _tcc_body,
        grid=(_NPAD // blk,),
        in_specs=[
            pl.BlockSpec((4, blk, _QP), lambda i: (0, i, 0)),
            pl.BlockSpec((1, oprev), lambda i: (0, 0)),
            pl.BlockSpec((oprev, out), lambda i: (0, 0)),
            pl.BlockSpec((1, out), lambda i: (0, 0)),
            pl.BlockSpec((1, out), lambda i: (0, 0)),
        ],
        out_specs=[
            pl.BlockSpec((blk, out), lambda i: (i, 0)),
            pl.BlockSpec((2, blk), lambda i: (0, i)),
        ],
        out_shape=[
            jax.ShapeDtypeStruct((_NPAD, out), jnp.float32),
            jax.ShapeDtypeStruct((2, _NPAD), jnp.float32),
        ],
    )(acc, b_prev[None], W, a_s[None], a_d[None])


def _tcf_body(acc_ref, b_ref, o_ref):
    msg = _quadrants(acc_ref, 2) + b_ref[0, :]
    mx = jnp.max(msg, axis=1, keepdims=True)
    t = msg - mx
    o_ref[...] = t - jnp.log(jnp.sum(jnp.exp(t), axis=1, keepdims=True))


def _tc_final(acc, b):
    blk = 400
    out = b.shape[0]
    return _pallas_call(
        _tcf_body,
        grid=(_N // blk,),
        in_specs=[
            pl.BlockSpec((4, blk, _QP), lambda i: (0, i, 0)),
            pl.BlockSpec((1, out), lambda i: (0, 0)),
        ],
        out_specs=pl.BlockSpec((blk, out), lambda i: (i, 0)),
        out_shape=jax.ShapeDtypeStruct((_N, out), jnp.float32),
    )(acc, b[None])


def _sc_gat(h4, a2, s2, d2):
    mesh = plsc.VectorSubcoreMesh(core_axis_name="c", subcore_axis_name="s")

    @functools.partial(
        _pl_kernel,
        out_type=jax.ShapeDtypeStruct((4, _NPAD, _QP), jnp.float32),
        mesh=mesh,
        compiler_params=pltpu.CompilerParams(
            needs_layout_passes=False, use_tc_tiling_on_sc=False),
        scratch_types=[
            pltpu.VMEM((_NCHH, _K), jnp.int32),
            pltpu.VMEM((_NCHH, _K), jnp.int32),
            pltpu.VMEM((_EPT,), jnp.float32),
            pltpu.VMEM((_NPAD,), jnp.float32),
            pltpu.VMEM((_NPAD,), jnp.float32),
            pltpu.VMEM((16,), jnp.float32),
            pltpu.VMEM((16, 16), jnp.float32),
            pltpu.VMEM((2, _K, _Q), jnp.float32),
            pltpu.VMEM((2, _K, _QP), jnp.float32),
            pltpu.VMEM((_ZR, _QP), jnp.float32),
            pltpu.VMEM_SHARED((_NPAD, _QP), jnp.float32),
            pltpu.VMEM_SHARED((16, 16), jnp.float32),
            pltpu.SemaphoreType.DMA,
            pltpu.SemaphoreType.DMA,
            pltpu.SemaphoreType.DMA,
            pltpu.SemaphoreType.DMA,
        ],
    )
    def k(h_hbm, a2_hbm, s_hbm, d_hbm, acc_hbm,
          sidx_v, didx_v, ebuf, av, bv, svec, mball, gbuf, hbuf,
          zrow, acc_sh, m_sh, sem0, sem1, ssem0, ssem1):
        cid = lax.axis_index("c")
        sid = lax.axis_index("s")

        pltpu.sync_copy(a2_hbm.at[0], av)
        pltpu.sync_copy(a2_hbm.at[1], bv)

        def stage_half(hh):
            pltpu.sync_copy(s_hbm.at[sid, pl.ds(hh * _NCHH, _NCHH)], sidx_v)
            pltpu.sync_copy(d_hbm.at[sid, pl.ds(hh * _NCHH, _NCHH)], didx_v)

        zero = jnp.zeros((16,), jnp.float32)
        for r in range(_ZR):
            for c in range(_QP // 16):
                zrow[r, pl.ds(c * 16, 16)] = zero

        def zero_acc():
            for t in range(_RPT // _ZR):
                pltpu.sync_copy(zrow,
                                acc_sh.at[pl.ds(sid * _RPT + t * _ZR, _ZR)])

        zero_acc()

        stage_half(0)

        def score_body(j, macc):
            m = macc
            for g in range(_K // 16):
                sv = sidx_v[j, pl.ds(g * 16, 16)]
                dv = didx_v[j, pl.ds(g * 16, 16)]
                t = plsc.load_gather(av, [sv]) + plsc.load_gather(bv, [dv])
                e = jnp.maximum(t, 0.2 * t)
                ebuf[pl.ds(j * _K + g * 16, 16)] = e
                sidx_v[j, pl.ds(g * 16, 16)] = sv * 4 + 2 * cid
                m = jnp.maximum(m, e)
            return m

        macc = lax.fori_loop(0, _NCH, score_body,
                             jnp.full((16,), -1e30, jnp.float32))

        svec[...] = macc
        pltpu.sync_copy(svec, m_sh.at[sid])
        plsc.subcore_barrier()
        pltpu.sync_copy(m_sh, mball)
        v = mball[0, :]
        for r in range(1, 16):
            v = jnp.maximum(v, mball[r, :])
        svec[...] = plsc.cummax(v)
        msp = plsc.load_gather(svec, [jnp.full((16,), 15, jnp.int32)])

        def exp_body(g, carry):
            e = ebuf[pl.ds(g * 16, 16)]
            ebuf[pl.ds(g * 16, 16)] = jnp.exp(e - msp)
            return carry

        lax.fori_loop(0, _EPT // 16, exp_body, 0)

        def issue_gather(j, b, sem):
            pltpu.async_copy(h_hbm.at[sidx_v.at[j]], gbuf.at[b], sem)

        lane0 = lax.iota(jnp.int32, 16) == 0

        def wait_scatter(b, ssem):
            pltpu.make_async_copy(hbuf.at[b], acc_sh.at[pl.ds(0, _K)],
                                  ssem).wait()

        def bump_body(j, carry):
            for g in range(_K // 16):
                sl = pl.ds(g * 16, 16)
                sidx_v[j, sl] = sidx_v[j, sl] + 1
            return carry

        for p in range(2):
            if p == 1:
                lax.fori_loop(0, _NCH, bump_body, 0)
                zero_acc()
                plsc.subcore_barrier()
            q = 2 * cid + p

            def agg_body(jj, carry):
                for b in range(2):
                    j = jj * 2 + b
                    sem = sem0 if b == 0 else sem1
                    ssem = ssem0 if b == 0 else ssem1
                    pltpu.make_async_copy(h_hbm.at[pl.ds(0, _K)],
                                          gbuf.at[b], sem).wait()

                    @pl.when(j >= 2)
                    def _():
                        wait_scatter(b, ssem)

                    for r in range(_K):
                        zr = plsc.load_gather(
                            ebuf, [jnp.full((16,), j * _K + r, jnp.int32)])
                        for c in range(_Q // 16):
                            hbuf[b, r, pl.ds(c * 16, 16)] = (
                                gbuf[b, r, pl.ds(c * 16, 16)] * zr)
                        hbuf[b, r, pl.ds(_Q, 16)] = jnp.where(
                            lane0, zr, 0.0)
                    pltpu.async_copy(hbuf.at[b], acc_sh.at[didx_v.at[j]],
                                     ssem, add=True)

                    @pl.when(j + 2 < _NCH)
                    def _():
                        issue_gather(j + 2, b, sem)
                return carry

            issue_gather(0, 0, sem0)
            issue_gather(1, 1, sem1)
            lax.fori_loop(0, _NCH // 2, agg_body, 0)
            wait_scatter(0, ssem0)
            wait_scatter(1, ssem1)
            plsc.subcore_barrier()
            pltpu.sync_copy(acc_sh.at[pl.ds(sid * _RPT, _RPT)],
                            acc_hbm.at[q, pl.ds(sid * _RPT, _RPT)])

    return k(h4, a2, s2, d2)


def kernel(x, edge_index, W0, b0, as0, ad0, W1, b1, as1, ad1,
           W2, b2, as2, ad2):
    src = edge_index[0].astype(jnp.int32)
    dst = edge_index[1].astype(jnp.int32)
    loop = jnp.arange(_N, dtype=jnp.int32)
    pad = jnp.full((_EPAD - _E - _N,), _N, jnp.int32)
    s_all = jnp.concatenate([src, loop, pad])
    d_all = jnp.concatenate([dst, loop, pad])
    s2 = s_all.reshape(16, _NCH, _K)
    d2 = d_all.reshape(16, _NCH, _K)
    xp = jnp.zeros((_NPAD, _D), jnp.float32).at[:_N].set(x)

    W2p = jnp.zeros((_D, _D), jnp.float32).at[:, :W2.shape[1]].set(W2)
    as2p = jnp.zeros((_D,), jnp.float32).at[:as2.shape[0]].set(as2)
    ad2p = jnp.zeros((_D,), jnp.float32).at[:ad2.shape[0]].set(ad2)
    Ws = jnp.stack([W1, W2p, W2p])
    bs = jnp.stack([b0, b1, b1])
    ass = jnp.stack([as1, as2p, as2p])
    ads = jnp.stack([ad1, ad2p, ad2p])

    h, a2 = _tc0(xp, W0, as0, ad0)

    def body(carry, ws):
        h, a2, _ = carry
        acc = _sc_gat(h.reshape(4 * _NPAD, _Q), a2, s2, d2)
        W, b, a_s, a_d = ws
        h2, a22 = _tc_comb(acc, b, W, a_s, a_d)
        return (h2, a22, acc), None

    acc0 = jnp.zeros((4, _NPAD, _QP), jnp.float32)
    (_, _, acc), _ = lax.scan(body, (h, a2, acc0), (Ws, bs, ass, ads))
    return _tc_final(acc, b2)
